# baseline (speedup 1.0000x reference)
import functools
import math

import jax
import jax.numpy as jnp
from jax.experimental import pallas as pl
from jax.experimental.pallas import tpu as pltpu

_SC = 1024


def _attn_body(q_ref, k_ref, v_ref, o_ref, acc_ref, m_ref, l_ref,
               *, S, SC, Qn, KVH, G, D):
    s = pl.program_id(1)
    nc = pl.num_programs(1)

    @pl.when(s == 0)
    def _init():
        m_ref[...] = jnp.full_like(m_ref, -1e30)
        l_ref[...] = jnp.zeros_like(l_ref)
        acc_ref[...] = jnp.zeros_like(acc_ref)

    rows = jax.lax.broadcasted_iota(jnp.int32, (Qn * G, SC), 0)
    cols = jax.lax.broadcasted_iota(jnp.int32, (Qn * G, SC), 1) + s * SC
    qpos = (S - Qn) + rows // G
    keep = cols <= qpos

    inv_sqrt_d = 1.0 / math.sqrt(D)

    for h in range(KVH):
        km = k_ref[0, :, h * D:(h + 1) * D]
        vm = v_ref[0, :, h * D:(h + 1) * D]

        ks = jnp.maximum(jnp.max(jnp.abs(km), axis=1, keepdims=True) / 7.0,
                         1e-8)
        kq = jnp.clip(jnp.round(km / ks), -8.0, 7.0) * ks
        vs = jnp.maximum(jnp.max(jnp.abs(vm), axis=1, keepdims=True) / 7.0,
                         1e-8)
        vq = jnp.clip(jnp.round(vm / vs), -8.0, 7.0) * vs

        qm = q_ref[0, h]
        scores = jax.lax.dot_general(
            qm, kq, (((1,), (1,)), ((), ())),
            preferred_element_type=jnp.float32) * inv_sqrt_d
        scores = jnp.where(keep, scores, jnp.float32(-1e30))

        m_old = m_ref[h]
        m_new = jnp.maximum(m_old, jnp.max(scores, axis=1, keepdims=True))
        alpha = jnp.exp(m_old - m_new)
        p = jnp.exp(scores - m_new)
        l_ref[h] = l_ref[h] * alpha + jnp.sum(p, axis=1, keepdims=True)
        acc_ref[h] = acc_ref[h] * alpha + jax.lax.dot_general(
            p, vq, (((1,), (0,)), ((), ())),
            preferred_element_type=jnp.float32)
        m_ref[h] = m_new

    @pl.when(s == nc - 1)
    def _emit():
        o_ref[0] = acc_ref[...] / l_ref[...]


def kernel(q, k, v, block_table):
    B, Qn, H, D = q.shape
    _, S, KVH, _ = k.shape
    G = H // KVH
    SC = _SC

    qg = (q.reshape(B, Qn, KVH, G, D)
           .transpose(0, 2, 1, 3, 4)
           .reshape(B, KVH, Qn * G, D))

    kf = k.reshape(B, S, KVH * D)
    vf = v.reshape(B, S, KVH * D)

    out = pl.pallas_call(
        functools.partial(_attn_body, S=S, SC=SC, Qn=Qn, KVH=KVH, G=G, D=D),
        grid=(B, S // SC),
        in_specs=[
            pl.BlockSpec((1, KVH, Qn * G, D), lambda b, s: (b, 0, 0, 0)),
            pl.BlockSpec((1, SC, KVH * D), lambda b, s: (b, s, 0)),
            pl.BlockSpec((1, SC, KVH * D), lambda b, s: (b, s, 0)),
        ],
        out_specs=pl.BlockSpec((1, KVH, Qn * G, D), lambda b, s: (b, 0, 0, 0)),
        out_shape=jax.ShapeDtypeStruct((B, KVH, Qn * G, D), jnp.float32),
        scratch_shapes=[
            pltpu.VMEM((KVH, Qn * G, D), jnp.float32),
            pltpu.VMEM((KVH, Qn * G, 1), jnp.float32),
            pltpu.VMEM((KVH, Qn * G, 1), jnp.float32),
        ],
        compiler_params=pltpu.CompilerParams(
            dimension_semantics=("parallel", "arbitrary"),
        ),
    )(qg, kf, vf)

    return (out.reshape(B, KVH, Qn, G, D)
               .transpose(0, 2, 1, 3, 4)
               .reshape(B, Qn, H, D))


# speedup vs baseline: 1.3986x; 1.3986x over previous
import functools
import math

import jax
import jax.numpy as jnp
from jax.experimental import pallas as pl
from jax.experimental.pallas import tpu as pltpu

_SC = 512


def _fake_quant(x):
    s = jnp.maximum(jnp.max(jnp.abs(x), axis=1, keepdims=True) / 7.0, 1e-8)
    return jnp.clip(jnp.round(x / s), -8.0, 7.0) * s


def _attn_body(q_ref, k_ref, v_ref, o_ref, acc_ref, m_ref, l_ref,
               *, S, SC, Qn, KVH, G, D):
    s = pl.program_id(1)
    nc = pl.num_programs(1)

    @pl.when(s == 0)
    def _init():
        m_ref[...] = jnp.full_like(m_ref, -1e30)
        l_ref[...] = jnp.zeros_like(l_ref)
        acc_ref[...] = jnp.zeros_like(acc_ref)

    kq = _fake_quant(k_ref[0].reshape(SC * KVH, D))
    vq = _fake_quant(v_ref[0].reshape(SC * KVH, D))
    kt = jnp.swapaxes(kq.reshape(SC, KVH, D), 0, 1)
    vt = jnp.swapaxes(vq.reshape(SC, KVH, D), 0, 1)

    rows = jax.lax.broadcasted_iota(jnp.int32, (Qn * G, SC), 0)
    cols = jax.lax.broadcasted_iota(jnp.int32, (Qn * G, SC), 1) + s * SC
    qpos = (S - Qn) + rows // G
    keep = cols <= qpos

    inv_sqrt_d = 1.0 / math.sqrt(D)

    for h in range(KVH):
        qm = q_ref[0, h]
        scores = jax.lax.dot_general(
            qm, kt[h], (((1,), (1,)), ((), ())),
            preferred_element_type=jnp.float32) * inv_sqrt_d
        scores = jnp.where(keep, scores, jnp.float32(-1e30))

        m_old = m_ref[h]
        m_new = jnp.maximum(m_old, jnp.max(scores, axis=1, keepdims=True))
        alpha = jnp.exp(m_old - m_new)
        p = jnp.exp(scores - m_new)
        l_ref[h] = l_ref[h] * alpha + jnp.sum(p, axis=1, keepdims=True)
        acc_ref[h] = acc_ref[h] * alpha + jax.lax.dot_general(
            p, vt[h], (((1,), (0,)), ((), ())),
            preferred_element_type=jnp.float32)
        m_ref[h] = m_new

    @pl.when(s == nc - 1)
    def _emit():
        o_ref[0] = acc_ref[...] / l_ref[...]


def kernel(q, k, v, block_table):
    B, Qn, H, D = q.shape
    _, S, KVH, _ = k.shape
    G = H // KVH
    SC = _SC

    qg = (q.reshape(B, Qn, KVH, G, D)
           .transpose(0, 2, 1, 3, 4)
           .reshape(B, KVH, Qn * G, D))

    out = pl.pallas_call(
        functools.partial(_attn_body, S=S, SC=SC, Qn=Qn, KVH=KVH, G=G, D=D),
        grid=(B, S // SC),
        in_specs=[
            pl.BlockSpec((1, KVH, Qn * G, D), lambda b, s: (b, 0, 0, 0)),
            pl.BlockSpec((1, SC, KVH, D), lambda b, s: (b, s, 0, 0)),
            pl.BlockSpec((1, SC, KVH, D), lambda b, s: (b, s, 0, 0)),
        ],
        out_specs=pl.BlockSpec((1, KVH, Qn * G, D), lambda b, s: (b, 0, 0, 0)),
        out_shape=jax.ShapeDtypeStruct((B, KVH, Qn * G, D), jnp.float32),
        scratch_shapes=[
            pltpu.VMEM((KVH, Qn * G, D), jnp.float32),
            pltpu.VMEM((KVH, Qn * G, 1), jnp.float32),
            pltpu.VMEM((KVH, Qn * G, 1), jnp.float32),
        ],
        compiler_params=pltpu.CompilerParams(
            dimension_semantics=("parallel", "arbitrary"),
        ),
    )(qg, k, v)

    return (out.reshape(B, KVH, Qn, G, D)
               .transpose(0, 2, 1, 3, 4)
               .reshape(B, Qn, H, D))


# speedup vs baseline: 2.9064x; 2.0780x over previous
import functools
import math

import jax
import jax.numpy as jnp
from jax.experimental import pallas as pl
from jax.experimental.pallas import tpu as pltpu


def _fake_quant(x):
    s = jnp.maximum(jnp.max(jnp.abs(x), axis=1, keepdims=True) / 7.0, 1e-8)
    return jnp.clip(jnp.round(x * (1.0 / s)), -8.0, 7.0) * s


def _attn_body(q_ref, k_hbm, v_hbm, o_ref,
               kbuf, vbuf, sem,
               *, S, Qn, B, KVH, G, D):
    i = pl.program_id(0)
    n = pl.num_programs(0)
    b, h = i // KVH, i % KVH

    def k_copy(slot, bb, hh):
        return pltpu.make_async_copy(
            k_hbm.at[bb, :, hh, :], kbuf.at[slot], sem.at[slot, 0])

    def v_copy(slot, bb, hh):
        return pltpu.make_async_copy(
            v_hbm.at[bb, :, hh, :], vbuf.at[slot], sem.at[slot, 1])

    slot = i % 2

    @pl.when(i == 0)
    def _prologue():
        k_copy(0, b, h).start()
        v_copy(0, b, h).start()

    @pl.when(i + 1 < n)
    def _prefetch():
        nb, nh = (i + 1) // KVH, (i + 1) % KVH
        k_copy(1 - slot, nb, nh).start()
        v_copy(1 - slot, nb, nh).start()

    k_copy(slot, b, h).wait()
    v_copy(slot, b, h).wait()

    km = kbuf[slot]
    vm = vbuf[slot]
    kq = _fake_quant(km)
    vq = _fake_quant(vm)

    qm = q_ref[0, 0]
    scores = jax.lax.dot_general(
        qm, kq, (((1,), (1,)), ((), ())),
        preferred_element_type=jnp.float32) * (1.0 / math.sqrt(D))

    rows = jax.lax.broadcasted_iota(jnp.int32, (Qn * G, S), 0)
    cols = jax.lax.broadcasted_iota(jnp.int32, (Qn * G, S), 1)
    qpos = (S - Qn) + rows // G
    scores = jnp.where(cols <= qpos, scores, jnp.float32(-1e30))

    m = jnp.max(scores, axis=1, keepdims=True)
    p = jnp.exp(scores - m)
    l = jnp.sum(p, axis=1, keepdims=True)
    o = jax.lax.dot_general(
        p, vq, (((1,), (0,)), ((), ())),
        preferred_element_type=jnp.float32)
    o_ref[0, 0] = o / l


def kernel(q, k, v, block_table):
    B, Qn, H, D = q.shape
    _, S, KVH, _ = k.shape
    G = H // KVH

    qg = (q.reshape(B, Qn, KVH, G, D)
           .transpose(0, 2, 1, 3, 4)
           .reshape(B, KVH, Qn * G, D))

    out = pl.pallas_call(
        functools.partial(_attn_body, S=S, Qn=Qn, B=B, KVH=KVH, G=G, D=D),
        grid=(B * KVH,),
        in_specs=[
            pl.BlockSpec((1, 1, Qn * G, D),
                         lambda i: (i // KVH, i % KVH, 0, 0)),
            pl.BlockSpec(memory_space=pl.ANY),
            pl.BlockSpec(memory_space=pl.ANY),
        ],
        out_specs=pl.BlockSpec((1, 1, Qn * G, D),
                               lambda i: (i // KVH, i % KVH, 0, 0)),
        out_shape=jax.ShapeDtypeStruct((B, KVH, Qn * G, D), jnp.float32),
        scratch_shapes=[
            pltpu.VMEM((2, S, D), jnp.float32),
            pltpu.VMEM((2, S, D), jnp.float32),
            pltpu.SemaphoreType.DMA((2, 2)),
        ],
        compiler_params=pltpu.CompilerParams(
            dimension_semantics=("arbitrary",),
        ),
    )(qg, k, v)

    return (out.reshape(B, KVH, Qn, G, D)
               .transpose(0, 2, 1, 3, 4)
               .reshape(B, Qn, H, D))


# speedup vs baseline: 2.9290x; 1.0078x over previous
import functools
import math

import jax
import jax.numpy as jnp
from jax.experimental import pallas as pl
from jax.experimental.pallas import tpu as pltpu


def _fake_quant(x):
    s = jnp.maximum(jnp.max(jnp.abs(x), axis=1, keepdims=True) / 7.0, 1e-8)
    return jnp.clip(jnp.round(x * (1.0 / s)), -8.0, 7.0) * s


def _attn_body(q_ref, k_hbm, v_hbm, o_ref,
               kbuf, vbuf, sem,
               *, S, Qn, B, KVH, G, D):
    i = pl.program_id(0)
    n = pl.num_programs(0)
    b, h = i // KVH, i % KVH

    H2 = S // 2

    def k_copy(slot, bb, hh, half):
        sl = pl.ds(half * H2, H2)
        return pltpu.make_async_copy(
            k_hbm.at[bb, sl, hh, :], kbuf.at[slot, sl], sem.at[slot, half])

    def v_copy(slot, bb, hh, half):
        sl = pl.ds(half * H2, H2)
        return pltpu.make_async_copy(
            v_hbm.at[bb, sl, hh, :], vbuf.at[slot, sl], sem.at[slot, 2 + half])

    def start_all(slot, bb, hh):
        k_copy(slot, bb, hh, 0).start()
        k_copy(slot, bb, hh, 1).start()
        v_copy(slot, bb, hh, 0).start()
        v_copy(slot, bb, hh, 1).start()

    slot = i % 2

    @pl.when(i == 0)
    def _prologue():
        start_all(0, b, h)

    @pl.when(i + 1 < n)
    def _prefetch():
        start_all(1 - slot, (i + 1) // KVH, (i + 1) % KVH)

    k_copy(slot, b, h, 0).wait()
    k_copy(slot, b, h, 1).wait()
    km = kbuf[slot]
    kq = _fake_quant(km)

    v_copy(slot, b, h, 0).wait()
    v_copy(slot, b, h, 1).wait()
    vm = vbuf[slot]
    vq = _fake_quant(vm)

    qm = q_ref[0, 0]
    scores = jax.lax.dot_general(
        qm, kq, (((1,), (1,)), ((), ())),
        preferred_element_type=jnp.float32) * (1.0 / math.sqrt(D))

    rows = jax.lax.broadcasted_iota(jnp.int32, (Qn * G, S), 0)
    cols = jax.lax.broadcasted_iota(jnp.int32, (Qn * G, S), 1)
    qpos = (S - Qn) + rows // G
    scores = jnp.where(cols <= qpos, scores, jnp.float32(-1e30))

    m = jnp.max(scores, axis=1, keepdims=True)
    p = jnp.exp(scores - m)
    l = jnp.sum(p, axis=1, keepdims=True)
    o = jax.lax.dot_general(
        p, vq, (((1,), (0,)), ((), ())),
        preferred_element_type=jnp.float32)
    o_ref[0, 0] = o / l


def kernel(q, k, v, block_table):
    B, Qn, H, D = q.shape
    _, S, KVH, _ = k.shape
    G = H // KVH

    qg = (q.reshape(B, Qn, KVH, G, D)
           .transpose(0, 2, 1, 3, 4)
           .reshape(B, KVH, Qn * G, D))

    out = pl.pallas_call(
        functools.partial(_attn_body, S=S, Qn=Qn, B=B, KVH=KVH, G=G, D=D),
        grid=(B * KVH,),
        in_specs=[
            pl.BlockSpec((1, 1, Qn * G, D),
                         lambda i: (i // KVH, i % KVH, 0, 0)),
            pl.BlockSpec(memory_space=pl.ANY),
            pl.BlockSpec(memory_space=pl.ANY),
        ],
        out_specs=pl.BlockSpec((1, 1, Qn * G, D),
                               lambda i: (i // KVH, i % KVH, 0, 0)),
        out_shape=jax.ShapeDtypeStruct((B, KVH, Qn * G, D), jnp.float32),
        scratch_shapes=[
            pltpu.VMEM((2, S, D), jnp.float32),
            pltpu.VMEM((2, S, D), jnp.float32),
            pltpu.SemaphoreType.DMA((2, 4)),
        ],
        compiler_params=pltpu.CompilerParams(
            dimension_semantics=("arbitrary",),
        ),
    )(qg, k, v)

    return (out.reshape(B, KVH, Qn, G, D)
               .transpose(0, 2, 1, 3, 4)
               .reshape(B, Qn, H, D))
